# baseline (device time: 102008 ns/iter reference)
import jax
import jax.numpy as jnp
from jax import lax
from jax.experimental import pallas as pl
from jax.experimental.pallas import tpu as pltpu

N_DEV = 4
F32 = jnp.float32
BF16 = jnp.bfloat16


def kernel(x, w_mat):
    m, k_per = x.shape
    _, n = w_mat.shape
    h2 = m // 4
    h4 = m // 8

    def body(x_ref, w_ref, dummy_ref, out_ref, work, stage, ru1, ru2, rv1, rv2, ss, rs, os):
        out_dmas = []
        my = lax.axis_index("i")
        yp = my ^ 1
        xp = my ^ 3
        a = my // 2
        b = (my ^ (my // 2)) & 1

        barrier_sem = pltpu.get_barrier_semaphore()
        for nbr in (yp, xp):
            pl.semaphore_signal(
                barrier_sem, inc=1,
                device_id=(nbr,), device_id_type=pl.DeviceIdType.MESH,
            )
        pl.semaphore_wait(barrier_sem, 2)

        u_keep = b * h2
        u_send = (1 - b) * h2
        uq_keep = u_keep + a * h4
        uq_send = u_keep + (1 - a) * h4
        v_base = m // 2
        v_keep = v_base + a * h2
        v_send = v_base + (1 - a) * h2
        vq_keep = v_keep + b * h4
        vq_send = v_keep + (1 - b) * h4

        w_bf = w_ref[:, :].astype(BF16)

        def gemm_rows(off, nrows):
            work[pl.ds(off, nrows), :] = jnp.dot(
                x_ref[pl.ds(off, nrows), :].astype(BF16),
                w_bf,
                preferred_element_type=F32,
            ).astype(BF16)

        def copy(src_rows, nrows, dst_ref, sem_idx, partner):
            return pltpu.make_async_remote_copy(
                src_ref=work.at[pl.ds(src_rows, nrows)],
                dst_ref=dst_ref,
                send_sem=ss.at[sem_idx],
                recv_sem=rs.at[sem_idx],
                device_id=(partner,),
                device_id_type=pl.DeviceIdType.MESH,
            )

        def reduce_rows(off, nrows, recv):
            cur = work[pl.ds(off, nrows), :].astype(F32)
            work[pl.ds(off, nrows), :] = (
                cur + recv[:, :].astype(F32)
            ).astype(BF16)

        def silu_rows(off, nrows):
            y = work[pl.ds(off, nrows), :].astype(F32)
            stage[pl.ds(off, nrows), :] = y * (1.0 / (1.0 + jnp.exp(-y)))
            dma = pltpu.make_async_copy(
                stage.at[pl.ds(off, nrows)],
                out_ref.at[pl.ds(off, nrows)],
                os.at[len(out_dmas)],
            )
            dma.start()
            out_dmas.append(dma)

        gemm_rows(u_send + (1 - a) * h4, h4)
        s1a = copy(u_send + (1 - a) * h4, h4, ru1.at[pl.ds((1 - a) * h4, h4)], 0, yp)
        s1a.start()
        gemm_rows(v_send + (1 - b) * h4, h4)
        t1a = copy(v_send + (1 - b) * h4, h4, rv1.at[pl.ds((1 - b) * h4, h4)], 6, xp)
        t1a.start()
        gemm_rows(u_send + a * h4, h4)
        s1b = copy(u_send + a * h4, h4, ru1.at[pl.ds(a * h4, h4)], 1, yp)
        s1b.start()
        gemm_rows(v_send + b * h4, h4)
        t1b = copy(v_send + b * h4, h4, rv1.at[pl.ds(b * h4, h4)], 7, xp)
        t1b.start()
        gemm_rows(u_keep, h2)
        gemm_rows(v_keep, h2)

        s1a.wait_recv()
        reduce_rows(uq_send, h4, ru1.at[pl.ds((1 - a) * h4, h4)])
        s2 = copy(uq_send, h4, ru2, 2, xp)
        s2.start()
        t1a.wait_recv()
        reduce_rows(vq_send, h4, rv1.at[pl.ds((1 - b) * h4, h4)])
        t2 = copy(vq_send, h4, rv2, 8, yp)
        t2.start()
        s1b.wait_recv()
        reduce_rows(uq_keep, h4, ru1.at[pl.ds(a * h4, h4)])
        t1b.wait_recv()
        reduce_rows(vq_keep, h4, rv1.at[pl.ds(b * h4, h4)])

        s2.wait_recv()
        reduce_rows(uq_keep, h4, ru2)
        s3 = copy(uq_keep, h4, work.at[pl.ds(uq_keep, h4)], 3, xp)
        s4a = copy(uq_keep, h4, work.at[pl.ds(uq_keep, h4)], 4, yp)
        s3.start()
        s4a.start()
        t2.wait_recv()
        reduce_rows(vq_keep, h4, rv2)
        t3 = copy(vq_keep, h4, work.at[pl.ds(vq_keep, h4)], 9, yp)
        t4a = copy(vq_keep, h4, work.at[pl.ds(vq_keep, h4)], 10, xp)
        t3.start()
        t4a.start()
        silu_rows(uq_keep, h4)
        silu_rows(vq_keep, h4)

        r3 = copy(uq_keep, h4, work.at[pl.ds(uq_send, h4)], 3, xp)
        r3.wait_recv()
        s4b = copy(uq_send, h4, work.at[pl.ds(uq_send, h4)], 5, yp)
        s4b.start()
        silu_rows(uq_send, h4)
        q3 = copy(vq_keep, h4, work.at[pl.ds(vq_send, h4)], 9, yp)
        q3.wait_recv()
        t4b = copy(vq_send, h4, work.at[pl.ds(vq_send, h4)], 11, xp)
        t4b.start()
        silu_rows(vq_send, h4)

        r4a = copy(uq_keep, h4, work.at[pl.ds(u_send + a * h4, h4)], 4, yp)
        q4a = copy(vq_keep, h4, work.at[pl.ds(v_send + b * h4, h4)], 10, xp)
        r4b = copy(uq_keep, h4, work.at[pl.ds(u_send + (1 - a) * h4, h4)], 5, yp)
        q4b = copy(vq_keep, h4, work.at[pl.ds(v_send + (1 - b) * h4, h4)], 11, xp)
        r4a.wait_recv()
        silu_rows(u_send + a * h4, h4)
        q4a.wait_recv()
        silu_rows(v_send + b * h4, h4)
        r4b.wait_recv()
        silu_rows(u_send + (1 - a) * h4, h4)
        q4b.wait_recv()
        silu_rows(v_send + (1 - b) * h4, h4)

        for d in (s1a, s1b, t1a, t1b, s2, t2, s3, s4a, t3, t4a, s4b, t4b):
            d.wait_send()
        for d in out_dmas:
            d.wait()

    dummy = jnp.zeros((m, n), F32)
    return pl.pallas_call(
        body,
        out_shape=jax.ShapeDtypeStruct((m, n), F32),
        in_specs=[
            pl.BlockSpec(memory_space=pltpu.VMEM),
            pl.BlockSpec(memory_space=pltpu.VMEM),
            pl.BlockSpec(memory_space=pl.ANY),
        ],
        out_specs=pl.BlockSpec(memory_space=pl.ANY),
        input_output_aliases={2: 0},
        scratch_shapes=[
            pltpu.VMEM((m, n), BF16),
            pltpu.VMEM((m, n), F32),
            pltpu.VMEM((h2, n), BF16),
            pltpu.VMEM((h4, n), BF16),
            pltpu.VMEM((h2, n), BF16),
            pltpu.VMEM((h4, n), BF16),
            pltpu.SemaphoreType.DMA((12,)),
            pltpu.SemaphoreType.DMA((12,)),
            pltpu.SemaphoreType.DMA((8,)),
        ],
        compiler_params=pltpu.CompilerParams(
            collective_id=0,
            vmem_limit_bytes=128 * 1024 * 1024,
        ),
    )(x, w_mat, dummy)


# device time: 95652 ns/iter; 1.0664x vs baseline; 1.0664x over previous
import jax
import jax.numpy as jnp
from jax import lax
from jax.experimental import pallas as pl
from jax.experimental.pallas import tpu as pltpu

N_DEV = 4
F32 = jnp.float32
BF16 = jnp.bfloat16


def kernel(x, w_mat):
    m, k_per = x.shape
    _, n = w_mat.shape
    h2 = m // 4
    h4 = m // 8

    def body(x_ref, w_ref, out_ref, work, stage, ru1, ru2, rv1, rv2, ss, rs, os):
        out_dmas = []
        my = lax.axis_index("i")
        yp = my ^ 1
        xp = my ^ 3
        a = my // 2
        b = (my ^ (my // 2)) & 1

        barrier_sem = pltpu.get_barrier_semaphore()
        for nbr in (yp, xp):
            pl.semaphore_signal(
                barrier_sem, inc=1,
                device_id=(nbr,), device_id_type=pl.DeviceIdType.MESH,
            )
        pl.semaphore_wait(barrier_sem, 2)

        u_keep = b * h2
        u_send = (1 - b) * h2
        uq_keep = u_keep + a * h4
        uq_send = u_keep + (1 - a) * h4
        v_base = m // 2
        v_keep = v_base + a * h2
        v_send = v_base + (1 - a) * h2
        vq_keep = v_keep + b * h4
        vq_send = v_keep + (1 - b) * h4

        w_bf = w_ref[:, :].astype(BF16)

        def gemm_rows(off, nrows):
            work[pl.ds(off, nrows), :] = jnp.dot(
                x_ref[pl.ds(off, nrows), :].astype(BF16),
                w_bf,
                preferred_element_type=F32,
            ).astype(BF16)

        def copy(src_rows, nrows, dst_ref, sem_idx, partner):
            return pltpu.make_async_remote_copy(
                src_ref=work.at[pl.ds(src_rows, nrows)],
                dst_ref=dst_ref,
                send_sem=ss.at[sem_idx],
                recv_sem=rs.at[sem_idx],
                device_id=(partner,),
                device_id_type=pl.DeviceIdType.MESH,
            )

        def reduce_rows(off, nrows, recv):
            cur = work[pl.ds(off, nrows), :].astype(F32)
            work[pl.ds(off, nrows), :] = (
                cur + recv[:, :].astype(F32)
            ).astype(BF16)

        def silu_rows(off, nrows):
            y = work[pl.ds(off, nrows), :].astype(F32)
            stage[pl.ds(off, nrows), :] = y * (1.0 / (1.0 + jnp.exp(-y)))
            dma = pltpu.make_async_copy(
                stage.at[pl.ds(off, nrows)],
                out_ref.at[pl.ds(off, nrows)],
                os.at[len(out_dmas)],
            )
            dma.start()
            out_dmas.append(dma)

        gemm_rows(u_send + (1 - a) * h4, h4)
        s1a = copy(u_send + (1 - a) * h4, h4, ru1.at[pl.ds((1 - a) * h4, h4)], 0, yp)
        s1a.start()
        gemm_rows(v_send + (1 - b) * h4, h4)
        t1a = copy(v_send + (1 - b) * h4, h4, rv1.at[pl.ds((1 - b) * h4, h4)], 6, xp)
        t1a.start()
        gemm_rows(u_send + a * h4, h4)
        s1b = copy(u_send + a * h4, h4, ru1.at[pl.ds(a * h4, h4)], 1, yp)
        s1b.start()
        gemm_rows(v_send + b * h4, h4)
        t1b = copy(v_send + b * h4, h4, rv1.at[pl.ds(b * h4, h4)], 7, xp)
        t1b.start()
        gemm_rows(u_keep, h2)
        gemm_rows(v_keep, h2)

        s1a.wait_recv()
        reduce_rows(uq_send, h4, ru1.at[pl.ds((1 - a) * h4, h4)])
        s2 = copy(uq_send, h4, ru2, 2, xp)
        s2.start()
        t1a.wait_recv()
        reduce_rows(vq_send, h4, rv1.at[pl.ds((1 - b) * h4, h4)])
        t2 = copy(vq_send, h4, rv2, 8, yp)
        t2.start()
        s1b.wait_recv()
        reduce_rows(uq_keep, h4, ru1.at[pl.ds(a * h4, h4)])
        t1b.wait_recv()
        reduce_rows(vq_keep, h4, rv1.at[pl.ds(b * h4, h4)])

        s2.wait_recv()
        reduce_rows(uq_keep, h4, ru2)
        s3 = copy(uq_keep, h4, work.at[pl.ds(uq_keep, h4)], 3, xp)
        s4a = copy(uq_keep, h4, work.at[pl.ds(uq_keep, h4)], 4, yp)
        s3.start()
        s4a.start()
        t2.wait_recv()
        reduce_rows(vq_keep, h4, rv2)
        t3 = copy(vq_keep, h4, work.at[pl.ds(vq_keep, h4)], 9, yp)
        t4a = copy(vq_keep, h4, work.at[pl.ds(vq_keep, h4)], 10, xp)
        t3.start()
        t4a.start()
        silu_rows(uq_keep, h4)
        silu_rows(vq_keep, h4)

        r3 = copy(uq_keep, h4, work.at[pl.ds(uq_send, h4)], 3, xp)
        r3.wait_recv()
        s4b = copy(uq_send, h4, work.at[pl.ds(uq_send, h4)], 5, yp)
        s4b.start()
        silu_rows(uq_send, h4)
        q3 = copy(vq_keep, h4, work.at[pl.ds(vq_send, h4)], 9, yp)
        q3.wait_recv()
        t4b = copy(vq_send, h4, work.at[pl.ds(vq_send, h4)], 11, xp)
        t4b.start()
        silu_rows(vq_send, h4)

        r4a = copy(uq_keep, h4, work.at[pl.ds(u_send + a * h4, h4)], 4, yp)
        q4a = copy(vq_keep, h4, work.at[pl.ds(v_send + b * h4, h4)], 10, xp)
        r4b = copy(uq_keep, h4, work.at[pl.ds(u_send + (1 - a) * h4, h4)], 5, yp)
        q4b = copy(vq_keep, h4, work.at[pl.ds(v_send + (1 - b) * h4, h4)], 11, xp)
        r4a.wait_recv()
        silu_rows(u_send + a * h4, h4)
        q4a.wait_recv()
        silu_rows(v_send + b * h4, h4)
        r4b.wait_recv()
        silu_rows(u_send + (1 - a) * h4, h4)
        q4b.wait_recv()
        silu_rows(v_send + (1 - b) * h4, h4)

        for d in (s1a, s1b, t1a, t1b, s2, t2, s3, s4a, t3, t4a, s4b, t4b):
            d.wait_send()
        for d in out_dmas:
            d.wait()

    return pl.pallas_call(
        body,
        out_shape=jax.ShapeDtypeStruct((m, n), F32),
        in_specs=[
            pl.BlockSpec(memory_space=pltpu.VMEM),
            pl.BlockSpec(memory_space=pltpu.VMEM),
        ],
        out_specs=pl.BlockSpec(memory_space=pl.ANY),
        scratch_shapes=[
            pltpu.VMEM((m, n), BF16),
            pltpu.VMEM((m, n), F32),
            pltpu.VMEM((h2, n), BF16),
            pltpu.VMEM((h4, n), BF16),
            pltpu.VMEM((h2, n), BF16),
            pltpu.VMEM((h4, n), BF16),
            pltpu.SemaphoreType.DMA((12,)),
            pltpu.SemaphoreType.DMA((12,)),
            pltpu.SemaphoreType.DMA((8,)),
        ],
        compiler_params=pltpu.CompilerParams(
            collective_id=0,
            vmem_limit_bytes=128 * 1024 * 1024,
        ),
    )(x, w_mat)


# device time: 95115 ns/iter; 1.0725x vs baseline; 1.0056x over previous
import jax
import jax.numpy as jnp
from jax import lax
from jax.experimental import pallas as pl
from jax.experimental.pallas import tpu as pltpu

N_DEV = 4
F32 = jnp.float32
BF16 = jnp.bfloat16


def kernel(x, w_mat):
    m, k_per = x.shape
    _, n = w_mat.shape
    h2 = m // 4
    h4 = m // 8
    h8 = m // 16

    def body(x_ref, w_ref, out_ref, work, stage, ru1, ru2, rv1, rv2, ss, rs, os):
        out_dmas = []
        my = lax.axis_index("i")
        yp = my ^ 1
        xp = my ^ 3
        a = my // 2
        b = (my ^ (my // 2)) & 1

        barrier_sem = pltpu.get_barrier_semaphore()
        for nbr in (yp, xp):
            pl.semaphore_signal(
                barrier_sem, inc=1,
                device_id=(nbr,), device_id_type=pl.DeviceIdType.MESH,
            )
        pl.semaphore_wait(barrier_sem, 2)

        u_keep = b * h2
        u_send = (1 - b) * h2
        uq_keep = u_keep + a * h4
        uq_send = u_keep + (1 - a) * h4
        v_base = m // 2
        v_keep = v_base + a * h2
        v_send = v_base + (1 - a) * h2
        vq_keep = v_keep + b * h4
        vq_send = v_keep + (1 - b) * h4

        w_bf = w_ref[:, :].astype(BF16)

        def gemm_rows(off, nrows):
            work[pl.ds(off, nrows), :] = jnp.dot(
                x_ref[pl.ds(off, nrows), :].astype(BF16),
                w_bf,
                preferred_element_type=F32,
            ).astype(BF16)

        def copy(src_rows, nrows, dst_ref, sem_idx, partner):
            return pltpu.make_async_remote_copy(
                src_ref=work.at[pl.ds(src_rows, nrows)],
                dst_ref=dst_ref,
                send_sem=ss.at[sem_idx],
                recv_sem=rs.at[sem_idx],
                device_id=(partner,),
                device_id_type=pl.DeviceIdType.MESH,
            )

        def reduce_rows(off, nrows, recv):
            cur = work[pl.ds(off, nrows), :].astype(F32)
            work[pl.ds(off, nrows), :] = (
                cur + recv[:, :].astype(F32)
            ).astype(BF16)

        def silu_rows(off, nrows):
            y = work[pl.ds(off, nrows), :].astype(F32)
            stage[pl.ds(off, nrows), :] = y * (1.0 / (1.0 + jnp.exp(-y)))
            dma = pltpu.make_async_copy(
                stage.at[pl.ds(off, nrows)],
                out_ref.at[pl.ds(off, nrows)],
                os.at[len(out_dmas)],
            )
            dma.start()
            out_dmas.append(dma)

        gemm_rows(u_send + (1 - a) * h4, h4)
        s1a = copy(u_send + (1 - a) * h4, h4, ru1.at[pl.ds((1 - a) * h4, h4)], 0, yp)
        s1a.start()
        gemm_rows(v_send + (1 - b) * h4, h4)
        t1a = copy(v_send + (1 - b) * h4, h4, rv1.at[pl.ds((1 - b) * h4, h4)], 6, xp)
        t1a.start()
        gemm_rows(u_send + a * h4, h4)
        s1b = copy(u_send + a * h4, h4, ru1.at[pl.ds(a * h4, h4)], 1, yp)
        s1b.start()
        gemm_rows(v_send + b * h4, h4)
        t1b = copy(v_send + b * h4, h4, rv1.at[pl.ds(b * h4, h4)], 7, xp)
        t1b.start()
        gemm_rows(u_keep, h2)
        gemm_rows(v_keep, h2)

        s1a.wait_recv()
        reduce_rows(uq_send, h4, ru1.at[pl.ds((1 - a) * h4, h4)])
        s2 = copy(uq_send, h4, ru2, 2, xp)
        s2.start()
        t1a.wait_recv()
        reduce_rows(vq_send, h4, rv1.at[pl.ds((1 - b) * h4, h4)])
        t2 = copy(vq_send, h4, rv2, 8, yp)
        t2.start()
        s1b.wait_recv()
        reduce_rows(uq_keep, h4, ru1.at[pl.ds(a * h4, h4)])
        t1b.wait_recv()
        reduce_rows(vq_keep, h4, rv1.at[pl.ds(b * h4, h4)])

        s2.wait_recv()
        reduce_rows(uq_keep, h4, ru2)
        s3 = copy(uq_keep, h4, work.at[pl.ds(uq_keep, h4)], 3, xp)
        s4a = copy(uq_keep, h4, work.at[pl.ds(uq_keep, h4)], 4, yp)
        s3.start()
        s4a.start()
        t2.wait_recv()
        reduce_rows(vq_keep, h4, rv2)
        t3 = copy(vq_keep, h4, work.at[pl.ds(vq_keep, h4)], 9, yp)
        t4a = copy(vq_keep, h4, work.at[pl.ds(vq_keep, h4)], 10, xp)
        t3.start()
        t4a.start()
        silu_rows(uq_keep, h4)
        silu_rows(vq_keep, h4)

        r3 = copy(uq_keep, h4, work.at[pl.ds(uq_send, h4)], 3, xp)
        r3.wait_recv()
        s4b1 = copy(uq_send, h8, work.at[pl.ds(uq_send, h8)], 5, yp)
        s4b2 = copy(uq_send + h8, h8, work.at[pl.ds(uq_send + h8, h8)], 12, yp)
        s4b1.start()
        s4b2.start()
        silu_rows(uq_send, h4)
        q3 = copy(vq_keep, h4, work.at[pl.ds(vq_send, h4)], 9, yp)
        q3.wait_recv()
        t4b1 = copy(vq_send, h8, work.at[pl.ds(vq_send, h8)], 11, xp)
        t4b2 = copy(vq_send + h8, h8, work.at[pl.ds(vq_send + h8, h8)], 13, xp)
        t4b1.start()
        t4b2.start()
        silu_rows(vq_send, h4)

        r4a = copy(uq_keep, h4, work.at[pl.ds(u_send + a * h4, h4)], 4, yp)
        q4a = copy(vq_keep, h4, work.at[pl.ds(v_send + b * h4, h4)], 10, xp)
        r4b1 = copy(uq_keep, h8, work.at[pl.ds(u_send + (1 - a) * h4, h8)], 5, yp)
        r4b2 = copy(uq_keep, h8, work.at[pl.ds(u_send + (1 - a) * h4 + h8, h8)], 12, yp)
        q4b1 = copy(vq_keep, h8, work.at[pl.ds(v_send + (1 - b) * h4, h8)], 11, xp)
        q4b2 = copy(vq_keep, h8, work.at[pl.ds(v_send + (1 - b) * h4 + h8, h8)], 13, xp)
        r4a.wait_recv()
        silu_rows(u_send + a * h4, h4)
        q4a.wait_recv()
        silu_rows(v_send + b * h4, h4)
        r4b1.wait_recv()
        silu_rows(u_send + (1 - a) * h4, h8)
        q4b1.wait_recv()
        silu_rows(v_send + (1 - b) * h4, h8)
        r4b2.wait_recv()
        silu_rows(u_send + (1 - a) * h4 + h8, h8)
        q4b2.wait_recv()
        silu_rows(v_send + (1 - b) * h4 + h8, h8)

        for d in (s1a, s1b, t1a, t1b, s2, t2, s3, s4a, t3, t4a,
                  s4b1, s4b2, t4b1, t4b2):
            d.wait_send()
        for d in out_dmas:
            d.wait()

    return pl.pallas_call(
        body,
        out_shape=jax.ShapeDtypeStruct((m, n), F32),
        in_specs=[
            pl.BlockSpec(memory_space=pltpu.VMEM),
            pl.BlockSpec(memory_space=pltpu.VMEM),
        ],
        out_specs=pl.BlockSpec(memory_space=pl.ANY),
        scratch_shapes=[
            pltpu.VMEM((m, n), BF16),
            pltpu.VMEM((m, n), F32),
            pltpu.VMEM((h2, n), BF16),
            pltpu.VMEM((h4, n), BF16),
            pltpu.VMEM((h2, n), BF16),
            pltpu.VMEM((h4, n), BF16),
            pltpu.SemaphoreType.DMA((14,)),
            pltpu.SemaphoreType.DMA((14,)),
            pltpu.SemaphoreType.DMA((10,)),
        ],
        compiler_params=pltpu.CompilerParams(
            collective_id=0,
            vmem_limit_bytes=128 * 1024 * 1024,
        ),
    )(x, w_mat)
